# Initial kernel scaffold; baseline (speedup 1.0000x reference)
import jax
import jax.numpy as jnp
from jax.experimental import pallas as pl


def kernel(node_features, edge_features, W1, b1, W2, b2, We, be, Wc, bc):
    raise NotImplementedError("write your pallas kernel here")


# speedup vs baseline: 394.3810x; 394.3810x over previous
import jax
import jax.numpy as jnp
from jax.experimental import pallas as pl

B, N, D_NODE = 16, 128, 128
HID1, HID2 = 128, 256
EDGE_HID = 128


def _gnn_kernel(nf_ref, ef_ref, W1_ref, b1_ref, W2_ref, b2_ref,
                We_ref, be_ref, Wcn_ref, Wce_ref, bc_ref, out_ref):
    m = jnp.mean(nf_ref[...], axis=1)
    e1 = jax.nn.relu(
        jnp.dot(m, W1_ref[...], preferred_element_type=jnp.float32)
        + b1_ref[...])
    e2 = jax.nn.relu(
        jnp.dot(e1, W2_ref[...], preferred_element_type=jnp.float32)
        + b2_ref[...])
    ee = jax.nn.relu(
        jnp.dot(ef_ref[...], We_ref[...], preferred_element_type=jnp.float32)
        + be_ref[...])
    out_ref[...] = (
        jnp.dot(e2, Wcn_ref[...], preferred_element_type=jnp.float32)
        + jnp.dot(ee, Wce_ref[...], preferred_element_type=jnp.float32)
        + bc_ref[...])


def kernel(node_features, edge_features, W1, b1, W2, b2, We, be, Wc, bc):
    b = node_features.shape[0]
    ef_flat = edge_features.reshape(b, -1)
    out = pl.pallas_call(
        _gnn_kernel,
        out_shape=jax.ShapeDtypeStruct((b, Wc.shape[1]), jnp.float32),
    )(node_features, ef_flat,
      W1, b1.reshape(1, -1), W2, b2.reshape(1, -1),
      We, be.reshape(1, -1),
      Wc[:HID2], Wc[HID2:], bc.reshape(1, -1))
    return out
